# Initial kernel scaffold; baseline (speedup 1.0000x reference)
import jax
import jax.numpy as jnp
from jax.experimental import pallas as pl


def kernel(features, img_feat, edge_index, params):
    raise NotImplementedError("write your pallas kernel here")


# speedup vs baseline: 1.0325x; 1.0325x over previous
import functools

import jax
import jax.numpy as jnp
from jax.experimental import pallas as pl
from jax.experimental.pallas import tpu as pltpu

N_BLOCK = 1024


def _proj_body(x_ref, w_ref, b_ref, o_ref):
    o_ref[...] = (
        jnp.dot(x_ref[...], w_ref[...], preferred_element_type=jnp.float32)
        + b_ref[...]
    )


def _proj(x, w, b):
    n, d_in = x.shape
    d_out = w.shape[1]
    grid = (pl.cdiv(n, N_BLOCK),)
    return pl.pallas_call(
        _proj_body,
        grid=grid,
        in_specs=[
            pl.BlockSpec((N_BLOCK, d_in), lambda i: (i, 0)),
            pl.BlockSpec((d_in, d_out), lambda i: (0, 0)),
            pl.BlockSpec((1, d_out), lambda i: (0, 0)),
        ],
        out_specs=pl.BlockSpec((N_BLOCK, d_out), lambda i: (i, 0)),
        out_shape=jax.ShapeDtypeStruct((n, d_out), jnp.float32),
    )(x, w, b.reshape(1, -1))


def _transformer_conv(x, edge_index, p):
    src = edge_index[0]
    dst = edge_index[1]
    n = x.shape[0]
    d = p["W_q"].shape[1]
    w_all = jnp.concatenate([p["W_q"], p["W_k"], p["W_v"], p["W_skip"]], axis=1)
    b_all = jnp.concatenate([p["b_q"], p["b_k"], p["b_v"], p["b_skip"]])
    qkvs = _proj(x, w_all, b_all)
    q = qkvs[:, :d]
    k = qkvs[:, d : 2 * d]
    v = qkvs[:, 2 * d : 3 * d]
    skip = qkvs[:, 3 * d :]
    alpha = jnp.sum(q[dst] * k[src], axis=-1) / jnp.sqrt(jnp.float32(d))
    amax = jax.ops.segment_max(alpha, dst, num_segments=n)
    amax = jnp.where(jnp.isfinite(amax), amax, 0.0)
    ex = jnp.exp(alpha - amax[dst])
    denom = jax.ops.segment_sum(ex, dst, num_segments=n)
    coef = ex / (denom[dst] + 1e-16)
    agg = jax.ops.segment_sum(coef[:, None] * v[src], dst, num_segments=n)
    return agg + skip


def kernel(features, img_feat, edge_index, params):
    elu = jax.nn.elu
    h1 = elu(_transformer_conv(features, edge_index, params["conv1"]))
    h2 = _transformer_conv(h1, edge_index, params["conv2"])
    h3 = elu(_transformer_conv(h2, edge_index, params["conv3"]))
    h4 = _transformer_conv(h3, edge_index, params["conv4"])
    img1 = elu(_transformer_conv(img_feat, edge_index, params["imgconv1"]))
    img2 = _transformer_conv(img1, edge_index, params["imgconv2"])
    img3 = elu(_transformer_conv(img2, edge_index, params["imgconv3"]))
    img4 = _transformer_conv(img3, edge_index, params["imgconv4"])
    concat = jnp.concatenate([h2, img2], axis=1)
    comb = elu(_transformer_conv(concat, edge_index, params["neck"]))
    c2 = _transformer_conv(comb, edge_index, params["neck2"])
    c3 = elu(_transformer_conv(c2, edge_index, params["c3"]))
    c4 = _transformer_conv(c3, edge_index, params["c4"])
    return (h2, img2, c2, h4, img4, c4)


# speedup vs baseline: 2.0024x; 1.9393x over previous
import functools
import math

import jax
import jax.numpy as jnp
from jax import lax
from jax.experimental import pallas as pl
from jax.experimental.pallas import tpu as pltpu
from jax.experimental.pallas import tpu_sc as plsc

N = 10000
E = 320000
P = 32
NPW = 320
N_PAD = P * NPW
NPW_A = 336
EPW = 11264
CA = 128
CE = 2000
N_BLOCK = 1024

_mesh = plsc.VectorSubcoreMesh(core_axis_name="c", subcore_axis_name="s")
_SC_PARAMS = pltpu.CompilerParams(
    needs_layout_passes=False, use_tc_tiling_on_sc=False)


def _wid():
    return lax.axis_index("s") * 2 + lax.axis_index("c")


def _lane_shuffle(x, idx):
    return lax.gather(
        x, idx[:, None],
        lax.GatherDimensionNumbers(
            offset_dims=(), collapsed_slice_dims=(0,), start_index_map=(0,)),
        (1,), mode=lax.GatherScatterMode.PROMISE_IN_BOUNDS)


def _group_body(src_hbm, dst_hbm, srcl_hbm, dstl_hbm, cnt_hbm,
                sbuf, dbuf, schunk, dchunk, cbuf):
    w = _wid()
    lo = w * NPW
    iota = lax.iota(jnp.int32, 16)
    zeros_i = jnp.zeros((16,), jnp.int32)
    pad_d = jnp.full((16,), NPW, jnp.int32)

    def init_body(i, _):
        sbuf[pl.ds(i * 16, 16)] = zeros_i
        dbuf[pl.ds(i * 16, 16)] = pad_d
        return 0

    lax.fori_loop(0, EPW // 16, init_body, 0)

    def chunk_body(c, ptr):
        pltpu.sync_copy(src_hbm.at[pl.ds(c * CE, CE)], schunk)
        pltpu.sync_copy(dst_hbm.at[pl.ds(c * CE, CE)], dchunk)

        def vreg_body(v, ptr):
            dv = dchunk[pl.ds(v * 16, 16)]
            sv = schunk[pl.ds(v * 16, 16)]
            dloc = dv - lo
            m = (dloc >= 0) & (dloc < NPW)
            csum = jnp.cumsum(jnp.where(m, 1, 0))
            pos = ptr + csum - 1
            plsc.store_scatter(dbuf, [pos], dloc, mask=m)
            plsc.store_scatter(sbuf, [pos], sv, mask=m)
            total = jnp.max(csum, axis=0)
            return jnp.minimum(ptr + total, EPW - 16)

        return lax.fori_loop(0, CE // 16, vreg_body, ptr)

    ptr = lax.fori_loop(0, E // CE, chunk_body, jnp.int32(0))

    cbuf[...] = jnp.full((16,), 1, jnp.int32) * ptr
    pltpu.sync_copy(cbuf, cnt_hbm.at[w])
    pltpu.sync_copy(sbuf, srcl_hbm.at[pl.ds(w * EPW, EPW)])
    pltpu.sync_copy(dbuf, dstl_hbm.at[pl.ds(w * EPW, EPW)])


@jax.jit
def _group_edges(src, dst):
    f = pl.kernel(
        _group_body,
        mesh=_mesh,
        compiler_params=_SC_PARAMS,
        out_type=(
            jax.ShapeDtypeStruct((P * EPW,), jnp.int32),
            jax.ShapeDtypeStruct((P * EPW,), jnp.int32),
            jax.ShapeDtypeStruct((P, 16), jnp.int32),
        ),
        scratch_types=[
            pltpu.VMEM((EPW,), jnp.int32),
            pltpu.VMEM((EPW,), jnp.int32),
            pltpu.VMEM((CE,), jnp.int32),
            pltpu.VMEM((CE,), jnp.int32),
            pltpu.VMEM((16,), jnp.int32),
        ],
    )
    return f(src, dst)


def _conv_body(q_hbm, k_hbm, v_hbm, skip_hbm, srcl_hbm, dstl_hbm, xout_hbm,
               qloc, src_v, dst_v, alpha_v, amax_v, den_v, kbuf, skipbuf,
               wbuf, sem, *, d, elu):
    w = _wid()
    nbase = w * NPW
    ebase = w * EPW
    iota = lax.iota(jnp.int32, 16)
    inv_sqrt_d = jnp.float32(1.0 / math.sqrt(d))
    zeros = jnp.zeros((16,), jnp.float32)
    neg = jnp.full((16,), -1e30, jnp.float32)

    pltpu.sync_copy(q_hbm.at[pl.ds(nbase, NPW)], qloc.at[pl.ds(0, NPW)])
    for r in range(NPW, NPW_A):
        for jj in range(0, d, 16):
            qloc.at[r][pl.ds(jj, 16)] = zeros
    pltpu.sync_copy(srcl_hbm.at[pl.ds(ebase, EPW)], src_v)
    pltpu.sync_copy(dstl_hbm.at[pl.ds(ebase, EPW)], dst_v)

    def acc_init(i, _):
        amax_v[pl.ds(i * 16, 16)] = neg
        den_v[pl.ds(i * 16, 16)] = zeros
        return 0

    lax.fori_loop(0, NPW_A // 16, acc_init, 0)

    def passa_chunk(c, _):
        pltpu.async_copy(
            k_hbm.at[src_v.at[pl.ds(c * CA, CA)]], kbuf, sem).wait()

        def passa_vreg(v, _):
            off = c * CA + v * 16
            dstv = dst_v[pl.ds(off, 16)]
            eloc = iota + v * 16
            acc = zeros
            for j in range(d):
                jv = jnp.full((16,), j, jnp.int32)
                qe = plsc.load_gather(qloc, [dstv, jv])
                ke = plsc.load_gather(kbuf, [eloc, jv])
                acc = acc + qe * ke
            alpha = acc * inv_sqrt_d
            alpha_v[pl.ds(off, 16)] = alpha
            m = alpha
            for s in range(1, 16):
                perm = (iota + s) & 15
                rd = _lane_shuffle(dstv, perm)
                rm = _lane_shuffle(alpha, perm)
                m = jnp.where(rd == dstv, jnp.maximum(m, rm), m)
            cur = plsc.load_gather(amax_v, [dstv])
            plsc.store_scatter(amax_v, [dstv], jnp.maximum(cur, m))
            return 0

        return lax.fori_loop(0, CA // 16, passa_vreg, 0)

    lax.fori_loop(0, EPW // CA, passa_chunk, 0)

    def passb(v, _):
        off = v * 16
        dstv = dst_v[pl.ds(off, 16)]
        am = plsc.load_gather(amax_v, [dstv])
        ex = jnp.exp(alpha_v[pl.ds(off, 16)] - am)
        alpha_v[pl.ds(off, 16)] = ex
        plsc.addupdate_scatter(den_v, [dstv], ex)
        return 0

    lax.fori_loop(0, EPW // 16, passb, 0)

    def out_init(i, _):
        for jj in range(0, d, 16):
            qloc.at[i][pl.ds(jj, 16)] = zeros
        return 0

    lax.fori_loop(0, NPW_A, out_init, 0)

    def passc_chunk(c, _):
        pltpu.async_copy(
            v_hbm.at[src_v.at[pl.ds(c * CA, CA)]], kbuf, sem).wait()

        def passc_vreg(v, _):
            off = c * CA + v * 16
            dstv = dst_v[pl.ds(off, 16)]
            eloc = iota + v * 16
            den = plsc.load_gather(den_v, [dstv])
            coef = alpha_v[pl.ds(off, 16)] / (den + 1e-16)
            for j in range(d):
                jv = jnp.full((16,), j, jnp.int32)
                ve = plsc.load_gather(kbuf, [eloc, jv])
                plsc.addupdate_scatter(qloc, [dstv, jv], coef * ve)
            return 0

        return lax.fori_loop(0, CA // 16, passc_vreg, 0)

    lax.fori_loop(0, EPW // CA, passc_chunk, 0)

    def epi_chunk(rc, _):
        pltpu.sync_copy(skip_hbm.at[pl.ds(nbase + rc * 64, 64)], skipbuf)

        def epi_row(r, _):
            for jj in range(0, d, 16):
                val = qloc.at[rc * 64 + r][pl.ds(jj, 16)] \
                    + skipbuf.at[r][pl.ds(jj, 16)]
                if elu:
                    val = jnp.where(val > 0, val, jnp.exp(val) - 1.0)
                wbuf.at[r][pl.ds(jj, 16)] = val
            return 0

        lax.fori_loop(0, 64, epi_row, 0)
        pltpu.sync_copy(wbuf, xout_hbm.at[pl.ds(nbase + rc * 64, 64)])
        return 0

    lax.fori_loop(0, NPW // 64, epi_chunk, 0)


@functools.lru_cache(maxsize=None)
def _make_conv(d, elu):
    return pl.kernel(
        functools.partial(_conv_body, d=d, elu=elu),
        mesh=_mesh,
        compiler_params=_SC_PARAMS,
        out_type=jax.ShapeDtypeStruct((N_PAD, d), jnp.float32),
        scratch_types=[
            pltpu.VMEM((NPW_A, d), jnp.float32),
            pltpu.VMEM((EPW,), jnp.int32),
            pltpu.VMEM((EPW,), jnp.int32),
            pltpu.VMEM((EPW,), jnp.float32),
            pltpu.VMEM((NPW_A,), jnp.float32),
            pltpu.VMEM((NPW_A,), jnp.float32),
            pltpu.VMEM((CA, d), jnp.float32),
            pltpu.VMEM((64, d), jnp.float32),
            pltpu.VMEM((64, d), jnp.float32),
            pltpu.SemaphoreType.DMA,
        ],
    )


def _proj_body(x_ref, w_ref, b_ref, q_ref, k_ref, v_ref, s_ref):
    o = (jnp.dot(x_ref[...], w_ref[...], preferred_element_type=jnp.float32)
         + b_ref[...])
    d = o.shape[1] // 4
    q_ref[...] = o[:, :d]
    k_ref[...] = o[:, d:2 * d]
    v_ref[...] = o[:, 2 * d:3 * d]
    s_ref[...] = o[:, 3 * d:]


def _proj(x, p):
    d_in = x.shape[1]
    d = p["W_q"].shape[1]
    w_all = jnp.concatenate([p["W_q"], p["W_k"], p["W_v"], p["W_skip"]], axis=1)
    b_all = jnp.concatenate([p["b_q"], p["b_k"], p["b_v"], p["b_skip"]])
    grid = (N_PAD // N_BLOCK,)
    out = pl.pallas_call(
        _proj_body,
        grid=grid,
        in_specs=[
            pl.BlockSpec((N_BLOCK, d_in), lambda i: (i, 0)),
            pl.BlockSpec((d_in, 4 * d), lambda i: (0, 0)),
            pl.BlockSpec((1, 4 * d), lambda i: (0, 0)),
        ],
        out_specs=[pl.BlockSpec((N_BLOCK, d), lambda i: (i, 0))] * 4,
        out_shape=[jax.ShapeDtypeStruct((N_PAD, d), jnp.float32)] * 4,
    )(x, w_all, b_all.reshape(1, -1))
    return out


def _sc_conv(x, srcl, dstl, p, elu):
    q, k, v, skip = _proj(x, p)
    d = q.shape[1]
    return _make_conv(d, elu)(q, k, v, skip, srcl, dstl)


def _xla_conv(x, edge_index, p, elu):
    src = edge_index[0]
    dst = edge_index[1]
    n = x.shape[0]
    d = p["W_q"].shape[1]
    q = x @ p["W_q"] + p["b_q"]
    k = x @ p["W_k"] + p["b_k"]
    v = x @ p["W_v"] + p["b_v"]
    skip = x @ p["W_skip"] + p["b_skip"]
    alpha = jnp.sum(q[dst] * k[src], axis=-1) / jnp.sqrt(jnp.float32(d))
    amax = jax.ops.segment_max(alpha, dst, num_segments=n)
    amax = jnp.where(jnp.isfinite(amax), amax, 0.0)
    ex = jnp.exp(alpha - amax[dst])
    denom = jax.ops.segment_sum(ex, dst, num_segments=n)
    coef = ex / (denom[dst] + 1e-16)
    agg = jax.ops.segment_sum(coef[:, None] * v[src], dst, num_segments=n)
    out = agg + skip
    return jax.nn.elu(out) if elu else out


_LAYERS = [
    ("conv1", True), ("conv2", False), ("conv3", True), ("conv4", False),
    ("imgconv1", True), ("imgconv2", False), ("imgconv3", True),
    ("imgconv4", False), ("neck", True), ("neck2", False),
    ("c3", True), ("c4", False),
]


def _stack(features, img_feat, params, conv):
    h1 = conv(features, params["conv1"], True)
    h2 = conv(h1, params["conv2"], False)
    h3 = conv(h2, params["conv3"], True)
    h4 = conv(h3, params["conv4"], False)
    img1 = conv(img_feat, params["imgconv1"], True)
    img2 = conv(img1, params["imgconv2"], False)
    img3 = conv(img2, params["imgconv3"], True)
    img4 = conv(img3, params["imgconv4"], False)
    concat = jnp.concatenate([h2, img2], axis=1)
    comb = conv(concat, params["neck"], True)
    c2 = conv(comb, params["neck2"], False)
    c3 = conv(c2, params["c3"], True)
    c4 = conv(c3, params["c4"], False)
    return (h2, img2, c2, h4, img4, c4)


def kernel(features, img_feat, edge_index, params):
    src = edge_index[0]
    dst = edge_index[1]
    srcl, dstl, cnt = _group_edges(src, dst)
    overflow = jnp.any(cnt[:, 0] >= EPW - 16)

    fpad = jnp.zeros((N_PAD, features.shape[1]), jnp.float32)
    fpad = fpad.at[:N].set(features)
    ipad = jnp.zeros((N_PAD, img_feat.shape[1]), jnp.float32)
    ipad = ipad.at[:N].set(img_feat)

    def sc_path(operand):
        fpad, ipad, srcl, dstl, params = operand
        outs = _stack(fpad, ipad, params,
                      lambda x, p, e: _sc_conv(x, srcl, dstl, p, e))
        return tuple(o[:N] for o in outs)

    def xla_path(operand):
        fpad, ipad, srcl, dstl, params = operand
        ei = jnp.stack([src, dst])
        outs = _stack(fpad[:N], ipad[:N], params,
                      lambda x, p, e: _xla_conv(x, ei, p, e))
        return outs

    return lax.cond(overflow, xla_path, sc_path,
                    (fpad, ipad, srcl, dstl, params))
